# Initial kernel scaffold; baseline (speedup 1.0000x reference)
import jax
import jax.numpy as jnp
from jax.experimental import pallas as pl


def kernel(x):
    raise NotImplementedError("write your pallas kernel here")


# speedup vs baseline: 15.4374x; 15.4374x over previous
import functools

import jax
import jax.numpy as jnp
from jax.experimental import pallas as pl
from jax.experimental.pallas import tpu as pltpu

K = 64
ROWS_PER_BLOCK = 16
N_COLS = 224 * 224


def _keys_from_f32(x):
    i = jax.lax.bitcast_convert_type(x, jnp.int32)
    return i ^ (jax.lax.shift_right_arithmetic(i, 31) & jnp.int32(0x7FFFFFFF))


def _topk_mean_body(x_ref, o_ref, keys_ref):
    x = x_ref[...]
    keys_ref[...] = _keys_from_f32(x)
    br = x.shape[0]

    def count_ge(cand):
        ge = keys_ref[...] >= cand
        return jnp.sum(ge.astype(jnp.int32), axis=1, keepdims=True)

    zero = jnp.zeros((br, 1), jnp.int32)
    int_min = jnp.full((br, 1), jnp.int32(-2147483648))
    p = jnp.where(count_ge(zero) >= K, zero, int_min)

    def bit_step(b, p):
        bit = jnp.int32(1) << (jnp.int32(30) - b)
        cand = p + bit
        return jnp.where(count_ge(cand) >= K, cand, p)

    p = jax.lax.fori_loop(0, 31, bit_step, p)

    gt = keys_ref[...] > p
    c_gt = jnp.sum(gt.astype(jnp.float32), axis=1, keepdims=True)
    s_gt = jnp.sum(jnp.where(gt, x, 0.0), axis=1, keepdims=True)
    t_val = jax.lax.bitcast_convert_type(
        p ^ (jax.lax.shift_right_arithmetic(p, 31) & jnp.int32(0x7FFFFFFF)),
        jnp.float32,
    )
    o_ref[...] = (s_gt + (jnp.float32(K) - c_gt) * t_val) * jnp.float32(1.0 / K)


@jax.jit
def kernel(x):
    b, c, h, w = x.shape
    n_rows = b * c
    x2 = x.reshape(n_rows, h * w)
    out = pl.pallas_call(
        _topk_mean_body,
        grid=(n_rows // ROWS_PER_BLOCK,),
        in_specs=[pl.BlockSpec((ROWS_PER_BLOCK, N_COLS), lambda i: (i, 0))],
        out_specs=pl.BlockSpec((ROWS_PER_BLOCK, 1), lambda i: (i, 0)),
        out_shape=jax.ShapeDtypeStruct((n_rows, 1), jnp.float32),
        scratch_shapes=[pltpu.VMEM((ROWS_PER_BLOCK, N_COLS), jnp.int32)],
    )(x2)
    return out.reshape(b, c, 1, 1)
